# baseline (speedup 1.0000x reference)
import functools

import jax
import jax.numpy as jnp
from jax import lax
from jax.experimental import pallas as pl
from jax.experimental.pallas import tpu as pltpu
from jax.experimental.pallas import tpu_sc as plsc

_SQRT_2_OVER_PI = 0.7978845608028654


def _gelu(x):
    u2 = (2.0 * _SQRT_2_OVER_PI) * (x + 0.044715 * (x * x * x))
    return x / (1.0 + jnp.exp(-u2))


_N = 10000
_D = 128
_K = 32
_H = 64
_MP = 16
_BN = 400
_BE = _BN * _K
_NS = 5
_SN = _N // _NS
_SE = _SN * _K
_SB = _SN // _BN


def _sc_gather(table, idx, dout, e, off, ch=80):
    d = table.shape[1]
    info = plsc.get_sparse_core_info()
    nw = info.num_cores * info.num_subcores
    bpw = e // nw
    grp = 5
    rows = ch * grp
    ngrp = bpw // rows
    assert bpw % rows == 0 and e % nw == 0 and off % 8 == 0
    mesh = plsc.VectorSubcoreMesh(core_axis_name="c", subcore_axis_name="s")
    scratch = [
        pltpu.VMEM((bpw,), jnp.int32),
        pltpu.VMEM((rows, d), jnp.float32),
        pltpu.VMEM((rows, d), jnp.float32),
        pltpu.SemaphoreType.DMA,
    ]
    if dout != d:
        scratch.append(pltpu.VMEM((rows, dout), jnp.float32))

    @functools.partial(
        pl.kernel,
        out_type=jax.ShapeDtypeStruct((e, dout), jnp.float32),
        mesh=mesh,
        scratch_types=scratch,
    )
    def gather_kernel(table_hbm, idx_hbm, out_hbm, idx_v, rows_a, rows_b, sem,
                      cmp_v=None):
        wid = lax.axis_index("s") * info.num_cores + lax.axis_index("c")
        base = wid * bpw
        pltpu.sync_copy(idx_hbm.at[pl.ds(off + base, bpw)], idx_v)

        def fire(g, buf):
            return [
                pltpu.async_copy(
                    table_hbm.at[idx_v.at[pl.ds(g * rows + j * ch, ch)]],
                    buf.at[pl.ds(j * ch, ch)],
                    sem,
                )
                for j in range(grp)
            ]

        def write(g, buf):
            if dout == d:
                pltpu.sync_copy(buf, out_hbm.at[pl.ds(base + g * rows, rows)])
            else:
                def crow(r, c):
                    cmp_v[r, :] = buf[r, :dout]
                    return c

                lax.fori_loop(0, rows, crow, 0)
                pltpu.sync_copy(cmp_v, out_hbm.at[pl.ds(base + g * rows, rows)])

        @pl.loop(0, ngrp - ngrp % 2, step=2)
        def _(t):
            ha = fire(t, rows_a)
            for h in ha:
                h.wait()
            hb = fire(t + 1, rows_b)
            write(t, rows_a)
            for h in hb:
                h.wait()
            write(t + 1, rows_b)

        if ngrp % 2:
            hl = fire(ngrp - 1, rows_a)
            for h in hl:
                h.wait()
            write(ngrp - 1, rows_a)

    return gather_kernel(table, idx)


def _lift(x, w, b):
    def body(x_r, w_r, b_r, o_r):
        o_r[...] = (
            jnp.dot(x_r[...], w_r[...], preferred_element_type=jnp.float32)
            + b_r[...]
        )

    return pl.pallas_call(
        body,
        grid=(_N // _BN,),
        in_specs=[
            pl.BlockSpec((_BN, _D), lambda i: (i, 0)),
            pl.BlockSpec((_D, _D), lambda i: (0, 0)),
            pl.BlockSpec((1, _D), lambda i: (0, 0)),
        ],
        out_specs=pl.BlockSpec((_BN, _D), lambda i: (i, 0)),
        out_shape=jax.ShapeDtypeStruct((_N, _D), jnp.float32),
    )(x, w, b.reshape(1, _D))


def _layer_slice(meshv, x, xg, mg, sl, w1a, w1b, b1, w2, b2, w3, b3, w, b,
                 pw, pb, last):
    def body(mesh_r, x_r, xg_r, mg_r, w1a_r, w1b_r, b1_r, w2_r, b2_r,
             w3_r, b3_r, w_r, b_r, pw_r, pb_r, o_r):
        a = (
            jnp.dot(mesh_r[...], w1a_r[...], preferred_element_type=jnp.float32)
            + b1_r[...]
        )
        bs = jnp.dot(mg_r[...], w1b_r[...], preferred_element_type=jnp.float32)
        h1 = _gelu(
            bs.reshape(_BN, _K, _H) + a[:, None, :]
        ).reshape(_BE, _H)
        h2 = _gelu(
            jnp.dot(h1, w2_r[...], preferred_element_type=jnp.float32) + b2_r[...]
        )
        kr = jnp.dot(h2, w3_r[...], preferred_element_type=jnp.float32) + b3_r[...]
        msg = kr * xg_r[...]
        agg = msg.reshape(_BN, _K, _D).sum(axis=1)
        xn = _gelu(
            jnp.dot(agg, w_r[...], preferred_element_type=jnp.float32)
            + b_r[...]
            + x_r[...]
        )
        if last:
            xn = (
                jnp.dot(xn, pw_r[...], preferred_element_type=jnp.float32)
                + pb_r[...]
            )
        o_r[...] = xn

    so = sl * _SB
    full = lambda shape: pl.BlockSpec(shape, lambda i: tuple(0 for _ in shape))
    return pl.pallas_call(
        body,
        grid=(_SB,),
        in_specs=[
            pl.BlockSpec((_BN, 3), lambda i, o=so: (o + i, 0)),
            pl.BlockSpec((_BN, _D), lambda i, o=so: (o + i, 0)),
            pl.BlockSpec((_BE, _D), lambda i: (i, 0)),
            pl.BlockSpec((_BE, _MP), lambda i: (i, 0)),
            full((3, _H)),
            full((_MP, _H)),
            full((1, _H)),
            full((_H, _H)),
            full((1, _H)),
            full((_H, _D)),
            full((1, _D)),
            full((_D, _D)),
            full((1, _D)),
            full((_D, _D)),
            full((1, _D)),
        ],
        out_specs=pl.BlockSpec((_BN, _D), lambda i: (i, 0)),
        out_shape=jax.ShapeDtypeStruct((_SN, _D), jnp.float32),
    )(meshv, x, xg, mg, w1a, w1b, b1, w2, b2, w3, b3, w, b, pw, pb)


def kernel(inp, params, mesh, src, dst):
    del dst
    mesh_tbl = jnp.pad(mesh, ((0, 0), (0, _D - 3)))
    mgs = [None] * _NS
    mgs[0] = _sc_gather(mesh_tbl, src, _MP, _SE, 0, ch=40)
    x = _lift(inp[0], params["lift_W"], params["lift_b"])
    pw = params["proj_W"]
    pb = params["proj_b"].reshape(1, _D)
    layers = params["layers"]
    for i, lp in enumerate(layers):
        wargs = (
            lp["kW1"][:3],
            jnp.pad(lp["kW1"][3:], ((0, _MP - 3), (0, 0))),
            lp["kb1"].reshape(1, _H),
            lp["kW2"], lp["kb2"].reshape(1, _H),
            lp["kW3"], lp["kb3"].reshape(1, _D),
            lp["W"] * (1.0 / _K), lp["b"].reshape(1, _D),
            pw, pb,
        )
        last = i == len(layers) - 1
        outs = []
        for s in range(_NS):
            xg = _sc_gather(x, src, _D, _SE, s * _SE)
            if i == 0 and s + 1 < _NS:
                mgs[s + 1] = _sc_gather(mesh_tbl, src, _MP, _SE,
                                        (s + 1) * _SE, ch=40)
            outs.append(
                _layer_slice(mesh, x, xg, mgs[s], s, *wargs, last=last)
            )
        x = jnp.concatenate(outs, axis=0)
    return x[None]


# speedup vs baseline: 1.0332x; 1.0332x over previous
import functools

import jax
import jax.numpy as jnp
from jax import lax
from jax.experimental import pallas as pl
from jax.experimental.pallas import tpu as pltpu
from jax.experimental.pallas import tpu_sc as plsc

_N = 10000
_D = 128
_K = 32
_H = 64
_MP = 16
_BN = 400
_BE = _BN * _K
_NS = 5
_SN = _N // _NS
_SE = _SN * _K
_SB = _SN // _BN


def _sc_gather(table, idx, dout, e, off, ch=80):
    d = table.shape[1]
    info = plsc.get_sparse_core_info()
    nw = info.num_cores * info.num_subcores
    bpw = e // nw
    grp = 5
    rows = ch * grp
    ngrp = bpw // rows
    assert bpw % rows == 0 and e % nw == 0 and off % 8 == 0
    mesh = plsc.VectorSubcoreMesh(core_axis_name="c", subcore_axis_name="s")
    scratch = [
        pltpu.VMEM((bpw,), jnp.int32),
        pltpu.VMEM((rows, d), jnp.float32),
        pltpu.VMEM((rows, d), jnp.float32),
        pltpu.SemaphoreType.DMA,
    ]
    if dout != d:
        scratch.append(pltpu.VMEM((rows, dout), jnp.float32))

    @functools.partial(
        pl.kernel,
        out_type=jax.ShapeDtypeStruct((e, dout), jnp.float32),
        mesh=mesh,
        scratch_types=scratch,
    )
    def gather_kernel(table_hbm, idx_hbm, out_hbm, idx_v, rows_a, rows_b, sem,
                      cmp_v=None):
        wid = lax.axis_index("s") * info.num_cores + lax.axis_index("c")
        base = wid * bpw
        pltpu.sync_copy(idx_hbm.at[pl.ds(off + base, bpw)], idx_v)

        def fire(g, buf):
            return [
                pltpu.async_copy(
                    table_hbm.at[idx_v.at[pl.ds(g * rows + j * ch, ch)]],
                    buf.at[pl.ds(j * ch, ch)],
                    sem,
                )
                for j in range(grp)
            ]

        def write(g, buf):
            if dout == d:
                pltpu.sync_copy(buf, out_hbm.at[pl.ds(base + g * rows, rows)])
            else:
                def crow(r, c):
                    cmp_v[r, :] = buf[r, :dout]
                    return c

                lax.fori_loop(0, rows, crow, 0)
                pltpu.sync_copy(cmp_v, out_hbm.at[pl.ds(base + g * rows, rows)])

        @pl.loop(0, ngrp - ngrp % 2, step=2)
        def _(t):
            ha = fire(t, rows_a)
            for h in ha:
                h.wait()
            hb = fire(t + 1, rows_b)
            write(t, rows_a)
            for h in hb:
                h.wait()
            write(t + 1, rows_b)

        if ngrp % 2:
            hl = fire(ngrp - 1, rows_a)
            for h in hl:
                h.wait()
            write(ngrp - 1, rows_a)

    return gather_kernel(table, idx)


def _lift(x, w, b):
    def body(x_r, w_r, b_r, o_r):
        o_r[...] = (
            jnp.dot(x_r[...], w_r[...], preferred_element_type=jnp.float32)
            + b_r[...]
        )

    return pl.pallas_call(
        body,
        grid=(_N // _BN,),
        in_specs=[
            pl.BlockSpec((_BN, _D), lambda i: (i, 0)),
            pl.BlockSpec((_D, _D), lambda i: (0, 0)),
            pl.BlockSpec((1, _D), lambda i: (0, 0)),
        ],
        out_specs=pl.BlockSpec((_BN, _D), lambda i: (i, 0)),
        out_shape=jax.ShapeDtypeStruct((_N, _D), jnp.float32),
    )(x, w, b.reshape(1, _D))


def _layer_slice(meshv, x, xg, mg, sl, w1a, w1b, b1, w2, b2, w3, b3, w, b,
                 pw, pb, last):
    def body(mesh_r, x_r, xg_r, mg_r, w1a_r, w1b_r, b1_r, w2_r, b2_r,
             w3_r, b3_r, w_r, b_r, pw_r, pb_r, o_r):
        a = (
            jnp.dot(mesh_r[...], w1a_r[...], preferred_element_type=jnp.float32)
            + b1_r[...]
        )
        bs = jnp.dot(mg_r[...], w1b_r[...], preferred_element_type=jnp.float32)
        h1 = jax.nn.gelu(
            bs.reshape(_BN, _K, _H) + a[:, None, :]
        ).reshape(_BE, _H)
        h2 = jax.nn.gelu(
            jnp.dot(h1, w2_r[...], preferred_element_type=jnp.float32) + b2_r[...]
        )
        kr = jnp.dot(h2, w3_r[...], preferred_element_type=jnp.float32) + b3_r[...]
        msg = kr * xg_r[...]
        agg = msg.reshape(_BN, _K, _D).sum(axis=1)
        xn = jax.nn.gelu(
            jnp.dot(agg, w_r[...], preferred_element_type=jnp.float32)
            + b_r[...]
            + x_r[...]
        )
        if last:
            xn = (
                jnp.dot(xn, pw_r[...], preferred_element_type=jnp.float32)
                + pb_r[...]
            )
        o_r[...] = xn

    so = sl * _SB
    full = lambda shape: pl.BlockSpec(shape, lambda i: tuple(0 for _ in shape))
    return pl.pallas_call(
        body,
        grid=(_SB,),
        in_specs=[
            pl.BlockSpec((_BN, 3), lambda i, o=so: (o + i, 0)),
            pl.BlockSpec((_BN, _D), lambda i, o=so: (o + i, 0)),
            pl.BlockSpec((_BE, _D), lambda i: (i, 0)),
            pl.BlockSpec((_BE, _MP), lambda i: (i, 0)),
            full((3, _H)),
            full((_MP, _H)),
            full((1, _H)),
            full((_H, _H)),
            full((1, _H)),
            full((_H, _D)),
            full((1, _D)),
            full((_D, _D)),
            full((1, _D)),
            full((_D, _D)),
            full((1, _D)),
        ],
        out_specs=pl.BlockSpec((_BN, _D), lambda i: (i, 0)),
        out_shape=jax.ShapeDtypeStruct((_SN, _D), jnp.float32),
    )(meshv, x, xg, mg, w1a, w1b, b1, w2, b2, w3, b3, w, b, pw, pb)


def kernel(inp, params, mesh, src, dst):
    del dst
    mesh_tbl = jnp.pad(mesh, ((0, 0), (0, _D - 3)))
    mgs = [None] * _NS
    mgs[0] = _sc_gather(mesh_tbl, src, _MP, _SE, 0, ch=40)
    x = _lift(inp[0], params["lift_W"], params["lift_b"])
    pw = params["proj_W"]
    pb = params["proj_b"].reshape(1, _D)
    layers = params["layers"]
    for i, lp in enumerate(layers):
        wargs = (
            lp["kW1"][:3],
            jnp.pad(lp["kW1"][3:], ((0, _MP - 3), (0, 0))),
            lp["kb1"].reshape(1, _H),
            lp["kW2"], lp["kb2"].reshape(1, _H),
            lp["kW3"], lp["kb3"].reshape(1, _D),
            lp["W"] * (1.0 / _K), lp["b"].reshape(1, _D),
            pw, pb,
        )
        last = i == len(layers) - 1
        outs = []
        for s in range(_NS):
            xg = _sc_gather(x, src, _D, _SE, s * _SE)
            if i == 0 and s + 1 < _NS:
                mgs[s + 1] = _sc_gather(mesh_tbl, src, _MP, _SE,
                                        (s + 1) * _SE, ch=40)
            outs.append(
                _layer_slice(mesh, x, xg, mgs[s], s, *wargs, last=last)
            )
        x = jnp.concatenate(outs, axis=0)
    return x[None]
